# baseline (speedup 1.0000x reference)
import functools

import jax
import jax.numpy as jnp
from jax import lax
from jax.experimental import pallas as pl
from jax.experimental.pallas import tpu as pltpu
from jax.experimental.pallas import tpu_sc as plsc

VOCAB = 33
HIDDEN = 1024
N_TAB = 6


@functools.partial(jax.jit, static_argnums=())
def _gather6(idx_flat, t0, t1, t2, t3, t4, t5):
    info = plsc.get_sparse_core_info()
    nw = info.num_cores * info.num_subcores
    num_cores = info.num_cores
    b = idx_flat.shape[0]
    b_per_w = b // nw
    chunk = 32
    n_chunks = b_per_w // chunk

    mesh = plsc.VectorSubcoreMesh(core_axis_name="c", subcore_axis_name="s")

    @functools.partial(
        pl.kernel,
        mesh=mesh,
        out_type=[jax.ShapeDtypeStruct((b, HIDDEN), jnp.float32)] * N_TAB,
        scratch_types=[
            pltpu.VMEM((chunk,), jnp.int32),
            pltpu.VMEM((chunk, HIDDEN), jnp.float32),
            pltpu.SemaphoreType.DMA,
        ],
    )
    def k(idx_hbm, tab0, tab1, tab2, tab3, tab4, tab5,
          out0, out1, out2, out3, out4, out5, idx_v, rows_v, sem):
        tabs = (tab0, tab1, tab2, tab3, tab4, tab5)
        outs = (out0, out1, out2, out3, out4, out5)
        wid = lax.axis_index("s") * num_cores + lax.axis_index("c")
        base = wid * b_per_w
        for t in range(N_TAB):
            def body(i, carry, _t=t):
                off = base + i * chunk
                pltpu.sync_copy(idx_hbm.at[pl.ds(off, chunk)], idx_v)
                pltpu.async_copy(tabs[_t].at[idx_v], rows_v, sem).wait()
                pltpu.sync_copy(rows_v, outs[_t].at[pl.ds(off, chunk)])
                return carry
            lax.fori_loop(0, n_chunks, body, 0)

    return k(idx_flat, t0, t1, t2, t3, t4, t5)


def kernel(inputs, table_0, table_1, table_2, table_3, table_4, table_5):
    shape = inputs.shape
    idx_flat = inputs.reshape(-1).astype(jnp.int32)
    outs = _gather6(idx_flat, table_0, table_1, table_2,
                    table_3, table_4, table_5)
    ve = [o.reshape(*shape, HIDDEN) for o in outs]
    ve = ve + ve[::-1]
    return tuple(ve)


# speedup vs baseline: 1.0141x; 1.0141x over previous
import functools

import jax
import jax.numpy as jnp
from jax import lax
from jax.experimental import pallas as pl
from jax.experimental.pallas import tpu as pltpu
from jax.experimental.pallas import tpu_sc as plsc

VOCAB = 33
HIDDEN = 1024
N_TAB = 6
CHUNK = 32


@jax.jit
def _gather6(idx2d, tabs):
    info = plsc.get_sparse_core_info()
    num_cores = info.num_cores
    nw = num_cores * info.num_subcores
    b = idx2d.shape[0] * idx2d.shape[1]
    b_per_w = b // nw
    n_chunks = b_per_w // CHUNK
    rows_per_w = b_per_w // CHUNK

    mesh = plsc.VectorSubcoreMesh(core_axis_name="c", subcore_axis_name="s")

    @functools.partial(
        pl.kernel,
        mesh=mesh,
        out_type=[jax.ShapeDtypeStruct((b, HIDDEN), jnp.float32)] * N_TAB,
        scratch_types=[
            pltpu.VMEM((rows_per_w, CHUNK), jnp.int32),
            pltpu.VMEM((CHUNK, HIDDEN), jnp.float32),
            pltpu.VMEM((CHUNK, HIDDEN), jnp.float32),
            pltpu.SemaphoreType.DMA,
            pltpu.SemaphoreType.DMA,
        ],
    )
    def k(idx_hbm, tabs_hbm, out0, out1, out2, out3, out4, out5,
          idx_v, buf0, buf1, sem0, sem1):
        outs = (out0, out1, out2, out3, out4, out5)
        bufs = (buf0, buf1)
        sems = (sem0, sem1)
        s_idx = lax.axis_index("s")
        c_idx = lax.axis_index("c")
        wid = s_idx * num_cores + c_idx
        base = wid * b_per_w

        pltpu.sync_copy(idx_hbm.at[pl.ds(wid * rows_per_w, rows_per_w)],
                        idx_v)

        def gather_start(tab, ck, bb):
            pltpu.async_copy(tab.at[idx_v.at[ck]], bufs[bb], sems[bb])

        def gather_wait(tab, ck, bb):
            pltpu.make_async_copy(tab.at[idx_v.at[ck]], bufs[bb],
                                  sems[bb]).wait()

        for t in range(N_TAB):
            tab = tabs_hbm.at[t]
            out = outs[t]

            for bb in range(2):
                gather_start(tab, bb, bb)

            def pair(g, carry, _tab=tab, _out=out):
                for bb in range(2):
                    ck = g * 2 + bb
                    gather_wait(_tab, ck, bb)
                    pltpu.sync_copy(bufs[bb],
                                    _out.at[pl.ds(base + ck * CHUNK, CHUNK)])
                    gather_start(_tab, ck + 2, bb)
                return carry

            lax.fori_loop(0, (n_chunks - 2) // 2, pair, 0)

            for bb in range(2):
                ck = n_chunks - 2 + bb
                gather_wait(tab, ck, bb)
                pltpu.sync_copy(bufs[bb],
                                out.at[pl.ds(base + ck * CHUNK, CHUNK)])

    return k(idx2d, tabs)


def kernel(inputs, table_0, table_1, table_2, table_3, table_4, table_5):
    shape = inputs.shape
    idx2d = inputs.reshape(-1, CHUNK).astype(jnp.int32)
    tabs = jnp.stack([table_0, table_1, table_2, table_3, table_4, table_5])
    outs = _gather6(idx2d, tabs)
    ve = [o.reshape(*shape, HIDDEN) for o in outs]
    ve = ve + ve[::-1]
    return tuple(ve)


# speedup vs baseline: 1.2494x; 1.2321x over previous
import functools

import jax
import jax.numpy as jnp
from jax import lax
from jax.experimental import pallas as pl
from jax.experimental.pallas import tpu as pltpu
from jax.experimental.pallas import tpu_sc as plsc

VOCAB = 33
HIDDEN = 1024
N_TAB = 6
CHUNK = 32


@jax.jit
def _gather6(idx2d, tabs):
    info = plsc.get_sparse_core_info()
    num_cores = info.num_cores
    nw = num_cores * info.num_subcores
    b = idx2d.shape[0] * idx2d.shape[1]
    b_per_w = b // nw
    n_chunks = b_per_w // CHUNK
    rows_per_w = b_per_w // CHUNK

    mesh = plsc.VectorSubcoreMesh(core_axis_name="c", subcore_axis_name="s")

    @functools.partial(
        pl.kernel,
        mesh=mesh,
        out_type=[jax.ShapeDtypeStruct((b, HIDDEN), jnp.float32)] * N_TAB,
        scratch_types=[
            pltpu.VMEM((rows_per_w, CHUNK), jnp.int32),
            pltpu.VMEM((CHUNK, HIDDEN), jnp.float32),
            pltpu.VMEM((CHUNK, HIDDEN), jnp.float32),
            pltpu.SemaphoreType.DMA,
            pltpu.SemaphoreType.DMA,
        ],
    )
    def k(idx_hbm, tabs_hbm, out0, out1, out2, out3, out4, out5,
          idx_v, buf0, buf1, sem0, sem1):
        outs = (out0, out1, out2, out3, out4, out5)
        bufs = (buf0, buf1)
        sems = (sem0, sem1)
        s_idx = lax.axis_index("s")
        c_idx = lax.axis_index("c")
        wid = s_idx * num_cores + c_idx
        base = wid * b_per_w

        pltpu.sync_copy(idx_hbm.at[pl.ds(wid * rows_per_w, rows_per_w)],
                        idx_v)

        def gather_start(tab, ck, bb):
            pltpu.async_copy(tab.at[idx_v.at[ck]], bufs[bb], sems[bb])

        def gather_wait(tab, ck, bb):
            pltpu.make_async_copy(tab.at[idx_v.at[ck]], bufs[bb],
                                  sems[bb]).wait()

        for t in range(N_TAB):
            tab = tabs_hbm.at[t]
            out = outs[t]

            for bb in range(2):
                gather_start(tab, bb, bb)

            def pair(g, carry, _tab=tab, _out=out):
                for bb in range(2):
                    ck = g * 2 + bb
                    gather_wait(_tab, ck, bb)
                    pltpu.sync_copy(bufs[bb],
                                    _out.at[pl.ds(base + ck * CHUNK, CHUNK)])
                    gather_start(_tab, ck + 2, bb)
                return carry

            lax.fori_loop(0, (n_chunks - 2) // 2, pair, 0)

            for bb in range(2):
                ck = n_chunks - 2 + bb
                gather_wait(tab, ck, bb)
                pltpu.sync_copy(bufs[bb],
                                out.at[pl.ds(base + ck * CHUNK, CHUNK)])

    return k(idx2d, tabs)


ROWS_BLK = 1024


def _tc_body(idx_ref, tabs_ref, *out_refs):
    idx = idx_ref[...]
    cols = jax.lax.broadcasted_iota(jnp.int32, (1, 64), 1)
    oh = (idx == cols).astype(jnp.float32)
    for t in range(N_TAB):
        out_refs[t][...] = jax.lax.dot_general(
            oh, tabs_ref[t],
            dimension_numbers=(((1,), (0,)), ((), ())),
            preferred_element_type=jnp.float32)


def _tc_gather6(idx_col, tabs_pad):
    b = idx_col.shape[0]
    grid = (b // ROWS_BLK,)
    return pl.pallas_call(
        _tc_body,
        grid=grid,
        in_specs=[
            pl.BlockSpec((ROWS_BLK, 1), lambda i: (i, 0)),
            pl.BlockSpec((N_TAB, 64, HIDDEN), lambda i: (0, 0, 0)),
        ],
        out_specs=[pl.BlockSpec((ROWS_BLK, HIDDEN), lambda i: (i, 0))
                   for _ in range(N_TAB)],
        out_shape=[jax.ShapeDtypeStruct((b, HIDDEN), jnp.float32)] * N_TAB,
    )(idx_col, tabs_pad)


def kernel(inputs, table_0, table_1, table_2, table_3, table_4, table_5):
    shape = inputs.shape
    idx_flat = inputs.reshape(-1).astype(jnp.int32)
    idx2d = idx_flat.reshape(-1, CHUNK)
    tabs = jnp.stack([table_0, table_1, table_2, table_3, table_4, table_5])
    outs = _gather6(idx2d, tabs)
    tabs_pad = jnp.pad(tabs, ((0, 0), (0, 64 - VOCAB), (0, 0)))
    dups = _tc_gather6(idx_flat.reshape(-1, 1), tabs_pad)
    ve = [o.reshape(*shape, HIDDEN) for o in outs]
    dv = [o.reshape(*shape, HIDDEN) for o in dups]
    return tuple(ve + dv[::-1])


# speedup vs baseline: 2.1504x; 1.7211x over previous
import functools

import jax
import jax.numpy as jnp
from jax import lax
from jax.experimental import pallas as pl
from jax.experimental.pallas import tpu as pltpu
from jax.experimental.pallas import tpu_sc as plsc

VOCAB = 33
HIDDEN = 1024
N_TAB = 6
CHUNK = 32


@jax.jit
def _gather6(idx2d, tabs):
    info = plsc.get_sparse_core_info()
    num_cores = info.num_cores
    nw = num_cores * info.num_subcores
    b = idx2d.shape[0] * idx2d.shape[1]
    b_per_w = b // nw
    n_chunks = b_per_w // CHUNK
    rows_per_w = b_per_w // CHUNK

    mesh = plsc.VectorSubcoreMesh(core_axis_name="c", subcore_axis_name="s")

    @functools.partial(
        pl.kernel,
        mesh=mesh,
        out_type=(
            [jax.ShapeDtypeStruct((b, HIDDEN), jnp.float32)] * N_TAB
            + [jax.ShapeDtypeStruct((nw, N_TAB, VOCAB, HIDDEN), jnp.float32)]
        ),
        scratch_types=[
            pltpu.VMEM((rows_per_w, CHUNK), jnp.int32),
            pltpu.VMEM((VOCAB, HIDDEN), jnp.float32),
            pltpu.VMEM((CHUNK, HIDDEN), jnp.float32),
            pltpu.VMEM((CHUNK, HIDDEN), jnp.float32),
            pltpu.SemaphoreType.DMA,
            pltpu.SemaphoreType.DMA,
        ],
    )
    def k(idx_hbm, tabs_hbm, out0, out1, out2, out3, out4, out5, reps,
          idx_v, tab_v, buf0, buf1, sem0, sem1):
        outs = (out0, out1, out2, out3, out4, out5)
        bufs = (buf0, buf1)
        sems = (sem0, sem1)
        s_idx = lax.axis_index("s")
        c_idx = lax.axis_index("c")
        wid = s_idx * num_cores + c_idx
        base = wid * b_per_w

        for t in range(N_TAB):
            pltpu.sync_copy(tabs_hbm.at[t], tab_v)
            pltpu.sync_copy(tab_v, reps.at[wid, t])

        pltpu.sync_copy(idx_hbm.at[pl.ds(wid * rows_per_w, rows_per_w)],
                        idx_v)

        def gather_start(tab, ck, bb):
            pltpu.async_copy(tab.at[idx_v.at[ck]], bufs[bb], sems[bb])

        def gather_wait(tab, ck, bb):
            pltpu.make_async_copy(tab.at[idx_v.at[ck]], bufs[bb],
                                  sems[bb]).wait()

        for t in range(N_TAB):
            tab = reps.at[wid, t]
            out = outs[t]

            for bb in range(2):
                gather_start(tab, bb, bb)

            def pair(g, carry, _tab=tab, _out=out):
                for bb in range(2):
                    ck = g * 2 + bb
                    gather_wait(_tab, ck, bb)
                    pltpu.sync_copy(bufs[bb],
                                    _out.at[pl.ds(base + ck * CHUNK, CHUNK)])
                    gather_start(_tab, ck + 2, bb)
                return carry

            lax.fori_loop(0, (n_chunks - 2) // 2, pair, 0)

            for bb in range(2):
                ck = n_chunks - 2 + bb
                gather_wait(tab, ck, bb)
                pltpu.sync_copy(bufs[bb],
                                out.at[pl.ds(base + ck * CHUNK, CHUNK)])

    return k(idx2d, tabs)[:N_TAB]


ROWS_BLK = 1024


def _tc_body(idx_ref, tabs_ref, *out_refs):
    idx = idx_ref[...]
    cols = jax.lax.broadcasted_iota(jnp.int32, (1, 64), 1)
    oh = (idx == cols).astype(jnp.float32)
    for t in range(N_TAB):
        out_refs[t][...] = jax.lax.dot_general(
            oh, tabs_ref[t],
            dimension_numbers=(((1,), (0,)), ((), ())),
            precision=jax.lax.Precision.HIGHEST,
            preferred_element_type=jnp.float32)


def _tc_gather6(idx_col, tabs_pad):
    b = idx_col.shape[0]
    grid = (b // ROWS_BLK,)
    return pl.pallas_call(
        _tc_body,
        grid=grid,
        in_specs=[
            pl.BlockSpec((ROWS_BLK, 1), lambda i: (i, 0)),
            pl.BlockSpec((N_TAB, 64, HIDDEN), lambda i: (0, 0, 0)),
        ],
        out_specs=[pl.BlockSpec((ROWS_BLK, HIDDEN), lambda i: (i, 0))
                   for _ in range(N_TAB)],
        out_shape=[jax.ShapeDtypeStruct((b, HIDDEN), jnp.float32)] * N_TAB,
    )(idx_col, tabs_pad)


def kernel(inputs, table_0, table_1, table_2, table_3, table_4, table_5):
    shape = inputs.shape
    idx_flat = inputs.reshape(-1).astype(jnp.int32)
    idx2d = idx_flat.reshape(-1, CHUNK)
    tabs = jnp.stack([table_0, table_1, table_2, table_3, table_4, table_5])
    outs = _gather6(idx2d, tabs)
    tabs_pad = jnp.pad(tabs, ((0, 0), (0, 64 - VOCAB), (0, 0)))
    dups = _tc_gather6(idx_flat.reshape(-1, 1), tabs_pad)
    ve = [o.reshape(*shape, HIDDEN) for o in outs]
    dv = [o.reshape(*shape, HIDDEN) for o in dups]
    return tuple(ve + dv[::-1])
